# Initial kernel scaffold; baseline (speedup 1.0000x reference)
import jax
import jax.numpy as jnp
from jax.experimental import pallas as pl


def kernel(point_features, labels, cluster_centers, training, W1, b1, W2, b2):
    raise NotImplementedError("write your pallas kernel here")


# speedup vs baseline: 5.0993x; 5.0993x over previous
import functools

import jax
import jax.numpy as jnp
from jax import lax
from jax.experimental import pallas as pl
from jax.experimental.pallas import tpu as pltpu
from jax.experimental.pallas import tpu_sc as plsc

N_POINTS = 320000
N_CENTERS = 10000
D_FEAT = 128
LPR = 128
NROWS = N_POINTS // LPR
NC = 2
NS = 16
NW = NC * NS
OCTS = -(-NROWS // 8)
NROWS_PAD = OCTS * 8
MAXR = 8 * (-(-OCTS // NW))
N_CENT_PAD = 10112
STRIPE = N_CENT_PAD // NS
CNT_W = 128


def _worker_range(wid):
    o0 = (wid * OCTS) // NW
    o1 = ((wid + 1) * OCTS) // NW
    start = 8 * o0
    n = jnp.minimum(8 * o1, NROWS) - start
    return start, n


def _sc_segment_sum(points, labels2d, zeros):
    mesh = plsc.VectorSubcoreMesh(core_axis_name="c", subcore_axis_name="s")

    @functools.partial(
        pl.kernel,
        out_type=jax.ShapeDtypeStruct((NC * N_CENT_PAD, D_FEAT), jnp.float32),
        mesh=mesh,
        scratch_types=[
            pltpu.VMEM((MAXR, LPR), jnp.int32),
            pltpu.VMEM((LPR, D_FEAT), jnp.float32),
            pltpu.VMEM((LPR, D_FEAT), jnp.float32),
            pltpu.VMEM_SHARED((N_CENT_PAD, D_FEAT), jnp.float32),
            pltpu.SemaphoreType.DMA,
            pltpu.SemaphoreType.DMA,
        ],
    )
    def k(points_hbm, labels_hbm, zeros_hbm, sums_hbm,
          idx_v, buf_a, buf_b, acc_s, sem_a, sem_b):
        cid = lax.axis_index("c")
        sid = lax.axis_index("s")
        wid = cid * NS + sid

        pltpu.sync_copy(zeros_hbm, acc_s.at[pl.ds(sid * STRIPE, STRIPE)])

        start, n = _worker_range(wid)
        pltpu.sync_copy(labels_hbm.at[pl.ds(start, MAXR)], idx_v)

        plsc.subcore_barrier()

        def body(i, _):
            i2 = 2 * i
            ra = jnp.minimum(start + i2, NROWS - 1)
            rb = jnp.minimum(start + i2 + 1, NROWS - 1)
            da = pltpu.async_copy(
                points_hbm.at[pl.ds(ra * LPR, LPR)], buf_a, sem_a)
            db = pltpu.async_copy(
                points_hbm.at[pl.ds(rb * LPR, LPR)], buf_b, sem_b)
            da.wait()

            @pl.when(i2 < n)
            def _():
                pltpu.sync_copy(buf_a, acc_s.at[idx_v.at[i2]], add=True)

            db.wait()

            @pl.when(i2 + 1 < n)
            def _():
                pltpu.sync_copy(buf_b, acc_s.at[idx_v.at[i2 + 1]], add=True)

            return _

        lax.fori_loop(0, MAXR // 2, body, None)

        plsc.subcore_barrier()

        base = cid * N_CENT_PAD + sid * STRIPE
        pltpu.sync_copy(acc_s.at[pl.ds(sid * STRIPE, STRIPE)],
                        sums_hbm.at[pl.ds(base, STRIPE)])

    return k(points, labels2d, zeros)


def _sc_segment_count(labels2d, zeros_cnt, ones):
    mesh = plsc.VectorSubcoreMesh(core_axis_name="c", subcore_axis_name="s")

    @functools.partial(
        pl.kernel,
        out_type=jax.ShapeDtypeStruct((NC * N_CENT_PAD, CNT_W), jnp.float32),
        mesh=mesh,
        scratch_types=[
            pltpu.VMEM((MAXR, LPR), jnp.int32),
            pltpu.VMEM((LPR, CNT_W), jnp.float32),
            pltpu.VMEM_SHARED((N_CENT_PAD, CNT_W), jnp.float32),
        ],
    )
    def k(labels_hbm, zeros_hbm, ones_hbm, counts_hbm,
          idx_v, ones_v, cnt_s):
        cid = lax.axis_index("c")
        sid = lax.axis_index("s")
        wid = cid * NS + sid

        pltpu.sync_copy(zeros_hbm, cnt_s.at[pl.ds(sid * STRIPE, STRIPE)])
        pltpu.sync_copy(ones_hbm, ones_v)

        start, n = _worker_range(wid)
        pltpu.sync_copy(labels_hbm.at[pl.ds(start, MAXR)], idx_v)

        plsc.subcore_barrier()

        def body(i, _):
            @pl.when(i < n)
            def _():
                pltpu.sync_copy(ones_v, cnt_s.at[idx_v.at[i]], add=True)

            return _

        lax.fori_loop(0, MAXR, body, None)

        plsc.subcore_barrier()

        base = cid * N_CENT_PAD + sid * STRIPE
        pltpu.sync_copy(cnt_s.at[pl.ds(sid * STRIPE, STRIPE)],
                        counts_hbm.at[pl.ds(base, STRIPE)])

    return k(labels2d, zeros_cnt, ones)


def _tc_mean_mlp(sums, counts, W1, b1, W2, b2):
    BLK = 1000
    grid = (N_CENTERS // BLK,)
    inv_sqrt2 = 0.7071067811865476

    def gelu_exact(x):
        return 0.5 * x * (1.0 + lax.erf(x * inv_sqrt2))

    def body(s_ref, c_ref, w1_ref, b1_ref, w2_ref, b2_ref, o_ref):
        s = s_ref[0] + s_ref[1]
        c = c_ref[0, :, 0:1] + c_ref[1, :, 0:1]
        mean = jnp.where(c > 0.0, s / jnp.maximum(c, 1.0), 0.0)
        h = jnp.dot(mean, w1_ref[...], preferred_element_type=jnp.float32)
        h = gelu_exact(h + b1_ref[...])
        o = jnp.dot(h, w2_ref[...], preferred_element_type=jnp.float32)
        o_ref[...] = gelu_exact(o + b2_ref[...])

    return pl.pallas_call(
        body,
        grid=grid,
        in_specs=[
            pl.BlockSpec((2, BLK, D_FEAT), lambda i: (0, i, 0)),
            pl.BlockSpec((2, BLK, CNT_W), lambda i: (0, i, 0)),
            pl.BlockSpec((D_FEAT, 300), lambda i: (0, 0)),
            pl.BlockSpec((1, 300), lambda i: (0, 0)),
            pl.BlockSpec((300, 300), lambda i: (0, 0)),
            pl.BlockSpec((1, 300), lambda i: (0, 0)),
        ],
        out_specs=pl.BlockSpec((BLK, 300), lambda i: (i, 0)),
        out_shape=jax.ShapeDtypeStruct((N_CENTERS, 300), jnp.float32),
    )(sums, counts, W1, b1, W2, b2)


@jax.jit
def _run(point_features, labels, cluster_centers, W1, b1, W2, b2):
    labels2d = labels.astype(jnp.int32).reshape(NROWS, LPR)
    labels2d = jnp.pad(labels2d, ((0, NROWS_PAD - NROWS), (0, 0)))
    zeros = jnp.zeros((STRIPE, D_FEAT), jnp.float32)
    zeros_cnt = jnp.zeros((STRIPE, CNT_W), jnp.float32)
    ones = jnp.ones((LPR, CNT_W), jnp.float32)
    counts = _sc_segment_count(labels2d, zeros_cnt, ones)
    sums = _sc_segment_sum(point_features, labels2d, zeros)
    sums = sums.reshape(NC, N_CENT_PAD, D_FEAT)
    counts = counts.reshape(NC, N_CENT_PAD, CNT_W)
    return _tc_mean_mlp(sums, counts, W1, b1.reshape(1, 300),
                        W2, b2.reshape(1, 300))


def kernel(point_features, labels, cluster_centers, training, W1, b1, W2, b2):
    return _run(point_features, labels, cluster_centers, W1, b1, W2, b2)


# speedup vs baseline: 5.6009x; 1.0984x over previous
import functools

import jax
import jax.numpy as jnp
from jax import lax
from jax.experimental import pallas as pl
from jax.experimental.pallas import tpu as pltpu
from jax.experimental.pallas import tpu_sc as plsc

N_POINTS = 320000
N_CENTERS = 10000
D_FEAT = 128
LPR = 128
NROWS = N_POINTS // LPR
NC = 2
NS = 16
NW = NC * NS
OCTS = -(-NROWS // 8)
NROWS_PAD = OCTS * 8
MAXR = 8 * (-(-OCTS // NW))
N_CENT_PAD = 10112
STRIPE = N_CENT_PAD // NS
CNT_W = 128


def _worker_range(wid):
    o0 = (wid * OCTS) // NW
    o1 = ((wid + 1) * OCTS) // NW
    start = 8 * o0
    n = jnp.minimum(8 * o1, NROWS) - start
    return start, n


def _sc_segment_sum(points, labels2d, zeros):
    mesh = plsc.VectorSubcoreMesh(core_axis_name="c", subcore_axis_name="s")

    @functools.partial(
        pl.kernel,
        out_type=jax.ShapeDtypeStruct((NC * N_CENT_PAD, D_FEAT), jnp.float32),
        mesh=mesh,
        scratch_types=[
            pltpu.VMEM((MAXR, LPR), jnp.int32),
            [pltpu.VMEM((LPR, D_FEAT), jnp.float32) for _ in range(2)],
            pltpu.VMEM_SHARED((N_CENT_PAD, D_FEAT), jnp.float32),
            [pltpu.SemaphoreType.DMA for _ in range(2)],
            [pltpu.SemaphoreType.DMA for _ in range(2)],
        ],
    )
    def k(points_hbm, labels_hbm, zeros_hbm, sums_hbm,
          idx_v, bufs, acc_s, dsem, ssem):
        cid = lax.axis_index("c")
        sid = lax.axis_index("s")
        wid = cid * NS + sid

        pltpu.sync_copy(zeros_hbm, acc_s.at[pl.ds(sid * STRIPE, STRIPE)])

        start, n = _worker_range(wid)
        pltpu.sync_copy(labels_hbm.at[pl.ds(start, MAXR)], idx_v)

        plsc.subcore_barrier()

        def dma_start(i, j):
            r = jnp.minimum(start + i, NROWS - 1)
            pltpu.async_copy(points_hbm.at[pl.ds(r * LPR, LPR)],
                             bufs[j], dsem[j])

        def dma_wait(j):
            pltpu.make_async_copy(points_hbm.at[pl.ds(0, LPR)],
                                  bufs[j], dsem[j]).wait()

        def scat_start(i, j):
            pltpu.async_copy(bufs[j], acc_s.at[idx_v.at[i]], ssem[j],
                             add=True)

        def scat_wait(j):
            pltpu.make_async_copy(bufs[j], acc_s.at[idx_v.at[0]],
                                  ssem[j]).wait()

        dma_start(0, 0)

        def body(k_, _):
            i = 2 * k_
            dma_wait(0)

            @pl.when(i < n)
            def _():
                scat_start(i, 0)

            @pl.when((k_ > 0) & (i - 1 < n))
            def _():
                scat_wait(1)

            dma_start(i + 1, 1)
            dma_wait(1)

            @pl.when(i + 1 < n)
            def _():
                scat_start(i + 1, 1)

            @pl.when(i < n)
            def _():
                scat_wait(0)

            dma_start(i + 2, 0)
            return _

        lax.fori_loop(0, MAXR // 2, body, None)

        @pl.when(MAXR - 1 < n)
        def _():
            scat_wait(1)

        dma_wait(0)

        plsc.subcore_barrier()

        base = cid * N_CENT_PAD + sid * STRIPE
        pltpu.sync_copy(acc_s.at[pl.ds(sid * STRIPE, STRIPE)],
                        sums_hbm.at[pl.ds(base, STRIPE)])

    return k(points, labels2d, zeros)


def _sc_segment_count(labels2d, zeros_cnt, ones):
    mesh = plsc.VectorSubcoreMesh(core_axis_name="c", subcore_axis_name="s")

    @functools.partial(
        pl.kernel,
        out_type=jax.ShapeDtypeStruct((NC * N_CENT_PAD, CNT_W), jnp.float32),
        mesh=mesh,
        scratch_types=[
            pltpu.VMEM((MAXR, LPR), jnp.int32),
            pltpu.VMEM((LPR, CNT_W), jnp.float32),
            pltpu.VMEM_SHARED((N_CENT_PAD, CNT_W), jnp.float32),
        ],
    )
    def k(labels_hbm, zeros_hbm, ones_hbm, counts_hbm,
          idx_v, ones_v, cnt_s):
        cid = lax.axis_index("c")
        sid = lax.axis_index("s")
        wid = cid * NS + sid

        pltpu.sync_copy(zeros_hbm, cnt_s.at[pl.ds(sid * STRIPE, STRIPE)])
        pltpu.sync_copy(ones_hbm, ones_v)

        start, n = _worker_range(wid)
        pltpu.sync_copy(labels_hbm.at[pl.ds(start, MAXR)], idx_v)

        plsc.subcore_barrier()

        def body(i, _):
            @pl.when(i < n)
            def _():
                pltpu.sync_copy(ones_v, cnt_s.at[idx_v.at[i]], add=True)

            return _

        lax.fori_loop(0, MAXR, body, None)

        plsc.subcore_barrier()

        base = cid * N_CENT_PAD + sid * STRIPE
        pltpu.sync_copy(cnt_s.at[pl.ds(sid * STRIPE, STRIPE)],
                        counts_hbm.at[pl.ds(base, STRIPE)])

    return k(labels2d, zeros_cnt, ones)


def _tc_mean_mlp(sums, counts, W1, b1, W2, b2):
    BLK = 1000
    grid = (N_CENTERS // BLK,)
    inv_sqrt2 = 0.7071067811865476

    def gelu_exact(x):
        return 0.5 * x * (1.0 + lax.erf(x * inv_sqrt2))

    def body(s_ref, c_ref, w1_ref, b1_ref, w2_ref, b2_ref, o_ref):
        s = s_ref[0] + s_ref[1]
        c = c_ref[0, :, 0:1] + c_ref[1, :, 0:1]
        mean = jnp.where(c > 0.0, s / jnp.maximum(c, 1.0), 0.0)
        h = jnp.dot(mean, w1_ref[...], preferred_element_type=jnp.float32)
        h = gelu_exact(h + b1_ref[...])
        o = jnp.dot(h, w2_ref[...], preferred_element_type=jnp.float32)
        o_ref[...] = gelu_exact(o + b2_ref[...])

    return pl.pallas_call(
        body,
        grid=grid,
        in_specs=[
            pl.BlockSpec((2, BLK, D_FEAT), lambda i: (0, i, 0)),
            pl.BlockSpec((2, BLK, CNT_W), lambda i: (0, i, 0)),
            pl.BlockSpec((D_FEAT, 300), lambda i: (0, 0)),
            pl.BlockSpec((1, 300), lambda i: (0, 0)),
            pl.BlockSpec((300, 300), lambda i: (0, 0)),
            pl.BlockSpec((1, 300), lambda i: (0, 0)),
        ],
        out_specs=pl.BlockSpec((BLK, 300), lambda i: (i, 0)),
        out_shape=jax.ShapeDtypeStruct((N_CENTERS, 300), jnp.float32),
    )(sums, counts, W1, b1, W2, b2)


@jax.jit
def _run(point_features, labels, cluster_centers, W1, b1, W2, b2):
    labels2d = labels.astype(jnp.int32).reshape(NROWS, LPR)
    labels2d = jnp.pad(labels2d, ((0, NROWS_PAD - NROWS), (0, 0)))
    zeros = jnp.zeros((STRIPE, D_FEAT), jnp.float32)
    zeros_cnt = jnp.zeros((STRIPE, CNT_W), jnp.float32)
    ones = jnp.ones((LPR, CNT_W), jnp.float32)
    counts = _sc_segment_count(labels2d, zeros_cnt, ones)
    sums = _sc_segment_sum(point_features, labels2d, zeros)
    sums = sums.reshape(NC, N_CENT_PAD, D_FEAT)
    counts = counts.reshape(NC, N_CENT_PAD, CNT_W)
    return _tc_mean_mlp(sums, counts, W1, b1.reshape(1, 300),
                        W2, b2.reshape(1, 300))


def kernel(point_features, labels, cluster_centers, training, W1, b1, W2, b2):
    return _run(point_features, labels, cluster_centers, W1, b1, W2, b2)


# speedup vs baseline: 7.3008x; 1.3035x over previous
import functools

import jax
import jax.numpy as jnp
from jax import lax
from jax.experimental import pallas as pl
from jax.experimental.pallas import tpu as pltpu
from jax.experimental.pallas import tpu_sc as plsc

N_POINTS = 320000
N_CENTERS = 10000
D_FEAT = 128
LPR = 128
NROWS = N_POINTS // LPR
NC = 2
NS = 16
NW = NC * NS
OCTS = -(-NROWS // 8)
NROWS_PAD = OCTS * 8
MAXR = 8 * (-(-OCTS // NW))
N_CENT_PAD = 10240
STRIPE = N_CENT_PAD // NS


def _worker_range(wid):
    o0 = (wid * OCTS) // NW
    o1 = ((wid + 1) * OCTS) // NW
    start = 8 * o0
    n = jnp.minimum(8 * o1, NROWS) - start
    return start, n


def _sc_segment_sum(points, labels2d, zeros):
    mesh = plsc.VectorSubcoreMesh(core_axis_name="c", subcore_axis_name="s")

    @functools.partial(
        pl.kernel,
        out_type=jax.ShapeDtypeStruct((NC * N_CENT_PAD, D_FEAT), jnp.float32),
        mesh=mesh,
        scratch_types=[
            pltpu.VMEM((MAXR, LPR), jnp.int32),
            [pltpu.VMEM((LPR, D_FEAT), jnp.float32) for _ in range(2)],
            pltpu.VMEM_SHARED((N_CENT_PAD, D_FEAT), jnp.float32),
            [pltpu.SemaphoreType.DMA for _ in range(2)],
            [pltpu.SemaphoreType.DMA for _ in range(2)],
        ],
    )
    def k(points_hbm, labels_hbm, zeros_hbm, sums_hbm,
          idx_v, bufs, acc_s, dsem, ssem):
        cid = lax.axis_index("c")
        sid = lax.axis_index("s")
        wid = cid * NS + sid

        pltpu.sync_copy(zeros_hbm, acc_s.at[pl.ds(sid * STRIPE, STRIPE)])

        start, n = _worker_range(wid)
        pltpu.sync_copy(labels_hbm.at[pl.ds(start, MAXR)], idx_v)

        plsc.subcore_barrier()

        def dma_start(i, j):
            r = jnp.minimum(start + i, NROWS - 1)
            pltpu.async_copy(points_hbm.at[pl.ds(r * LPR, LPR)],
                             bufs[j], dsem[j])

        def dma_wait(j):
            pltpu.make_async_copy(points_hbm.at[pl.ds(0, LPR)],
                                  bufs[j], dsem[j]).wait()

        def scat_start(i, j):
            pltpu.async_copy(bufs[j], acc_s.at[idx_v.at[i]], ssem[j],
                             add=True)

        def scat_wait(j):
            pltpu.make_async_copy(bufs[j], acc_s.at[idx_v.at[0]],
                                  ssem[j]).wait()

        dma_start(0, 0)

        def body(k_, _):
            i = 2 * k_
            dma_wait(0)
            scat_start(i, 0)

            @pl.when(k_ > 0)
            def _():
                scat_wait(1)

            dma_start(i + 1, 1)
            dma_wait(1)
            scat_start(i + 1, 1)
            scat_wait(0)
            dma_start(i + 2, 0)
            return _

        lax.fori_loop(0, n // 2, body, None)

        scat_wait(1)
        dma_wait(0)

        plsc.subcore_barrier()

        base = cid * N_CENT_PAD + sid * STRIPE
        pltpu.sync_copy(acc_s.at[pl.ds(sid * STRIPE, STRIPE)],
                        sums_hbm.at[pl.ds(base, STRIPE)])

    return k(points, labels2d, zeros)


def _sc_segment_count(labels2d, zeros1d):
    mesh = plsc.VectorSubcoreMesh(core_axis_name="c", subcore_axis_name="s")

    @functools.partial(
        pl.kernel,
        out_type=jax.ShapeDtypeStruct((NW * N_CENT_PAD,), jnp.float32),
        mesh=mesh,
        compiler_params=pltpu.CompilerParams(needs_layout_passes=False),
        scratch_types=[
            pltpu.VMEM((MAXR, LPR), jnp.int32),
            pltpu.VMEM((N_CENT_PAD,), jnp.float32),
        ],
    )
    def k(labels_hbm, zeros_hbm, counts_hbm, idx_v, cnt_v):
        cid = lax.axis_index("c")
        sid = lax.axis_index("s")
        wid = cid * NS + sid

        pltpu.sync_copy(zeros_hbm, cnt_v)
        start, n = _worker_range(wid)
        pltpu.sync_copy(labels_hbm.at[pl.ds(start, MAXR)], idx_v)

        ones16 = jnp.full((16,), 1.0, jnp.float32)

        def body(i, _):
            for j in range(LPR // 16):
                lab16 = idx_v[i, pl.ds(16 * j, 16)]
                plsc.addupdate_scatter(cnt_v, [lab16], ones16)
            return _

        lax.fori_loop(0, n, body, None)

        pltpu.sync_copy(cnt_v, counts_hbm.at[pl.ds(wid * N_CENT_PAD,
                                                   N_CENT_PAD)])

    return k(labels2d, zeros1d)


def _tc_mean_mlp(sums, counts3d, W1, b1, W2, b2):
    BLK = 1024
    BR = BLK // LPR
    grid = (N_CENT_PAD // BLK,)
    inv_sqrt2 = 0.7071067811865476

    def gelu_exact(x):
        return 0.5 * x * (1.0 + lax.erf(x * inv_sqrt2))

    def body(s_ref, c_ref, w1_ref, b1_ref, w2_ref, b2_ref, o_ref):
        s = s_ref[0] + s_ref[1]
        c8 = jnp.sum(c_ref[...], axis=0)
        seg = lax.broadcasted_iota(jnp.int32, (BLK, LPR), 0)
        lane = lax.broadcasted_iota(jnp.int32, (BLK, LPR), 1)
        seg_r = lax.broadcasted_iota(jnp.int32, (BLK, BR), 0)
        sub = lax.broadcasted_iota(jnp.int32, (BLK, BR), 1)
        pick_row = (seg_r // LPR == sub).astype(jnp.float32)
        spread = jnp.dot(pick_row, c8, preferred_element_type=jnp.float32)
        c = jnp.sum(jnp.where(seg % LPR == lane, spread, 0.0),
                    axis=1, keepdims=True)
        mean = jnp.where(c > 0.0, s / jnp.maximum(c, 1.0), 0.0)
        h = jnp.dot(mean, w1_ref[...], preferred_element_type=jnp.float32)
        h = gelu_exact(h + b1_ref[...])
        o = jnp.dot(h, w2_ref[...], preferred_element_type=jnp.float32)
        o_ref[...] = gelu_exact(o + b2_ref[...])

    return pl.pallas_call(
        body,
        grid=grid,
        in_specs=[
            pl.BlockSpec((2, BLK, D_FEAT), lambda i: (0, i, 0)),
            pl.BlockSpec((NW, BR, LPR), lambda i: (0, i, 0)),
            pl.BlockSpec((D_FEAT, 300), lambda i: (0, 0)),
            pl.BlockSpec((1, 300), lambda i: (0, 0)),
            pl.BlockSpec((300, 300), lambda i: (0, 0)),
            pl.BlockSpec((1, 300), lambda i: (0, 0)),
        ],
        out_specs=pl.BlockSpec((BLK, 300), lambda i: (i, 0)),
        out_shape=jax.ShapeDtypeStruct((N_CENTERS, 300), jnp.float32),
    )(sums, counts3d, W1, b1, W2, b2)


@jax.jit
def _run(point_features, labels, cluster_centers, W1, b1, W2, b2):
    labels2d = labels.astype(jnp.int32).reshape(NROWS, LPR)
    labels2d = jnp.pad(labels2d, ((0, NROWS_PAD - NROWS), (0, 0)))
    zeros = jnp.zeros((STRIPE, D_FEAT), jnp.float32)
    zeros1d = jnp.zeros((N_CENT_PAD,), jnp.float32)
    counts = _sc_segment_count(labels2d, zeros1d)
    sums = _sc_segment_sum(point_features, labels2d, zeros)
    sums = sums.reshape(NC, N_CENT_PAD, D_FEAT)
    counts3d = counts.reshape(NW, N_CENT_PAD // LPR, LPR)
    return _tc_mean_mlp(sums, counts3d, W1, b1.reshape(1, 300),
                        W2, b2.reshape(1, 300))


def kernel(point_features, labels, cluster_centers, training, W1, b1, W2, b2):
    return _run(point_features, labels, cluster_centers, W1, b1, W2, b2)


# speedup vs baseline: 7.7053x; 1.0554x over previous
import functools

import jax
import jax.numpy as jnp
from jax import lax
from jax.experimental import pallas as pl
from jax.experimental.pallas import tpu as pltpu
from jax.experimental.pallas import tpu_sc as plsc

N_POINTS = 320000
N_CENTERS = 10000
D_FEAT = 128
LPR = 128
NROWS = N_POINTS // LPR
NC = 2
NS = 16
NW = NC * NS
OCTS = -(-NROWS // 8)
NROWS_PAD = OCTS * 8
MAXR = 8 * (-(-OCTS // NW))
N_CENT_PAD = 10112
STRIPE = N_CENT_PAD // NS


def _worker_range(wid):
    o0 = (wid * OCTS) // NW
    o1 = ((wid + 1) * OCTS) // NW
    start = 8 * o0
    n = jnp.minimum(8 * o1, NROWS) - start
    return start, n


def _sc_segment_sum(points, labels1d, zeros):
    mesh = plsc.VectorSubcoreMesh(core_axis_name="c", subcore_axis_name="s")

    @functools.partial(
        pl.kernel,
        out_type=jax.ShapeDtypeStruct((NC * N_CENT_PAD, D_FEAT), jnp.float32),
        mesh=mesh,
        scratch_types=[
            [pltpu.VMEM((LPR,), jnp.int32) for _ in range(3)],
            [pltpu.VMEM((LPR, D_FEAT), jnp.float32) for _ in range(3)],
            pltpu.VMEM_SHARED((N_CENT_PAD, D_FEAT), jnp.float32),
            [pltpu.SemaphoreType.DMA for _ in range(3)],
            [pltpu.SemaphoreType.DMA for _ in range(3)],
            [pltpu.SemaphoreType.DMA for _ in range(3)],
        ],
    )
    def k(points_hbm, labels_hbm, zeros_hbm, sums_hbm,
          lidx, bufs, acc_s, lsem, dsem, ssem):
        cid = lax.axis_index("c")
        sid = lax.axis_index("s")
        wid = cid * NS + sid

        pltpu.sync_copy(zeros_hbm, acc_s.at[pl.ds(sid * STRIPE, STRIPE)])

        start, n = _worker_range(wid)

        plsc.subcore_barrier()

        def dma_start(i, j):
            r = jnp.minimum(start + i, NROWS - 1)
            pltpu.async_copy(labels_hbm.at[pl.ds(r * LPR, LPR)],
                             lidx[j], lsem[j])
            pltpu.async_copy(points_hbm.at[pl.ds(r * LPR, LPR)],
                             bufs[j], dsem[j])

        def dma_wait(j):
            pltpu.make_async_copy(labels_hbm.at[pl.ds(0, LPR)],
                                  lidx[j], lsem[j]).wait()
            pltpu.make_async_copy(points_hbm.at[pl.ds(0, LPR)],
                                  bufs[j], dsem[j]).wait()

        def scat_start(j):
            pltpu.async_copy(bufs[j], acc_s.at[lidx[j]], ssem[j], add=True)

        def scat_wait(j):
            pltpu.make_async_copy(bufs[j], acc_s.at[lidx[j]],
                                  ssem[j]).wait()

        dma_start(0, 0)
        dma_start(1, 1)
        t = n // 3
        tail = n - 3 * t

        def body(k_, _):
            i0 = 3 * k_

            def step(i, j, pj, first_guard):
                dma_wait(j)
                scat_start(j)
                if first_guard:
                    @pl.when(k_ > 0)
                    def _():
                        scat_wait(pj)
                else:
                    scat_wait(pj)
                dma_start(i + 2, pj)

            step(i0, 0, 2, True)
            step(i0 + 1, 1, 0, False)
            step(i0 + 2, 2, 1, False)
            return _

        lax.fori_loop(0, t, body, None)

        @pl.when(tail > 0)
        def _():
            dma_wait(0)
            scat_start(0)
            scat_wait(2)

        @pl.when(tail > 1)
        def _():
            dma_wait(1)
            scat_start(1)
            scat_wait(0)

        @pl.when(tail == 0)
        def _():
            dma_wait(0)

        @pl.when(tail < 2)
        def _():
            dma_wait(1)

        @pl.when(tail == 0)
        def _():
            scat_wait(2)

        @pl.when(tail == 1)
        def _():
            scat_wait(0)

        @pl.when(tail == 2)
        def _():
            scat_wait(1)

        plsc.subcore_barrier()

        base = cid * N_CENT_PAD + sid * STRIPE
        pltpu.sync_copy(acc_s.at[pl.ds(sid * STRIPE, STRIPE)],
                        sums_hbm.at[pl.ds(base, STRIPE)])

    return k(points, labels1d, zeros)


def _sc_segment_count(labels2d, zeros1d):
    mesh = plsc.VectorSubcoreMesh(core_axis_name="c", subcore_axis_name="s")

    @functools.partial(
        pl.kernel,
        out_type=jax.ShapeDtypeStruct((NW * N_CENT_PAD,), jnp.float32),
        mesh=mesh,
        compiler_params=pltpu.CompilerParams(needs_layout_passes=False),
        scratch_types=[
            pltpu.VMEM((MAXR, LPR), jnp.int32),
            pltpu.VMEM((N_CENT_PAD,), jnp.float32),
        ],
    )
    def k(labels_hbm, zeros_hbm, counts_hbm, idx_v, cnt_v):
        cid = lax.axis_index("c")
        sid = lax.axis_index("s")
        wid = cid * NS + sid

        pltpu.sync_copy(zeros_hbm, cnt_v)
        start, n = _worker_range(wid)
        pltpu.sync_copy(labels_hbm.at[pl.ds(start, MAXR)], idx_v)

        ones16 = jnp.full((16,), 1.0, jnp.float32)

        def body(i, _):
            for j in range(LPR // 16):
                lab16 = idx_v[i, pl.ds(16 * j, 16)]
                plsc.addupdate_scatter(cnt_v, [lab16], ones16)
            return _

        lax.fori_loop(0, n, body, None)

        pltpu.sync_copy(cnt_v, counts_hbm.at[pl.ds(wid * N_CENT_PAD,
                                                   N_CENT_PAD)])

    return k(labels2d, zeros1d)


def _tc_mean_mlp(sums, counts3d, W1, b1, W2, b2):
    BLK = 1024
    BR = BLK // LPR
    grid = (-(-N_CENTERS // BLK),)
    inv_sqrt2 = 0.7071067811865476

    def gelu_exact(x):
        return 0.5 * x * (1.0 + lax.erf(x * inv_sqrt2))

    def body(s_ref, c_ref, w1_ref, b1_ref, w2_ref, b2_ref, o_ref):
        s = s_ref[0] + s_ref[1]
        c8 = jnp.sum(c_ref[...], axis=0)
        seg = lax.broadcasted_iota(jnp.int32, (BLK, LPR), 0)
        lane = lax.broadcasted_iota(jnp.int32, (BLK, LPR), 1)
        seg_r = lax.broadcasted_iota(jnp.int32, (BLK, BR), 0)
        sub = lax.broadcasted_iota(jnp.int32, (BLK, BR), 1)
        pick_row = (seg_r // LPR == sub).astype(jnp.float32)
        spread = jnp.dot(pick_row, c8, preferred_element_type=jnp.float32)
        c = jnp.sum(jnp.where(seg % LPR == lane, spread, 0.0),
                    axis=1, keepdims=True)
        mean = jnp.where(c > 0.0, s / jnp.maximum(c, 1.0), 0.0)
        h = jnp.dot(mean, w1_ref[...], preferred_element_type=jnp.float32)
        h = gelu_exact(h + b1_ref[...])
        o = jnp.dot(h, w2_ref[...], preferred_element_type=jnp.float32)
        o_ref[...] = gelu_exact(o + b2_ref[...])

    return pl.pallas_call(
        body,
        grid=grid,
        in_specs=[
            pl.BlockSpec((2, BLK, D_FEAT), lambda i: (0, i, 0)),
            pl.BlockSpec((NW, BR, LPR), lambda i: (0, i, 0)),
            pl.BlockSpec((D_FEAT, 300), lambda i: (0, 0)),
            pl.BlockSpec((1, 300), lambda i: (0, 0)),
            pl.BlockSpec((300, 300), lambda i: (0, 0)),
            pl.BlockSpec((1, 300), lambda i: (0, 0)),
        ],
        out_specs=pl.BlockSpec((BLK, 300), lambda i: (i, 0)),
        out_shape=jax.ShapeDtypeStruct((N_CENTERS, 300), jnp.float32),
    )(sums, counts3d, W1, b1, W2, b2)


@jax.jit
def _run(point_features, labels, cluster_centers, W1, b1, W2, b2):
    labels2d = labels.astype(jnp.int32).reshape(NROWS, LPR)
    labels2d = jnp.pad(labels2d, ((0, NROWS_PAD - NROWS), (0, 0)))
    labels1d = labels2d.reshape(NROWS_PAD * LPR)
    zeros = jnp.zeros((STRIPE, D_FEAT), jnp.float32)
    zeros1d = jnp.zeros((N_CENT_PAD,), jnp.float32)
    counts = _sc_segment_count(labels2d, zeros1d)
    sums = _sc_segment_sum(point_features, labels1d, zeros)
    sums = sums.reshape(NC, N_CENT_PAD, D_FEAT)
    counts3d = counts.reshape(NW, N_CENT_PAD // LPR, LPR)
    cr = counts3d.shape[1]
    counts3d = jnp.pad(counts3d, ((0, 0), (0, -cr % 8), (0, 0)))
    return _tc_mean_mlp(sums, counts3d, W1, b1.reshape(1, 300),
                        W2, b2.reshape(1, 300))


def kernel(point_features, labels, cluster_centers, training, W1, b1, W2, b2):
    return _run(point_features, labels, cluster_centers, W1, b1, W2, b2)
